# Initial kernel scaffold; baseline (speedup 1.0000x reference)
import jax
import jax.numpy as jnp
from jax.experimental import pallas as pl


def kernel(X, edge_index, W1, b1, a1, W2, b2, Wg1, bg1, Wg2, bg2, ag):
    raise NotImplementedError("write your pallas kernel here")


# speedup vs baseline: 9.5808x; 9.5808x over previous
import functools

import jax
import jax.numpy as jnp
from jax import lax
from jax.experimental import pallas as pl
from jax.experimental.pallas import tpu as pltpu
from jax.experimental.pallas import tpu_sc as plsc

NC = 2
NS = 16
LB = 128


def _mesh():
    return plsc.VectorSubcoreMesh(
        core_axis_name="c", subcore_axis_name="s", num_cores=NC,
        num_subcores=NS)


def _make_deg(NPAD, CH):
    CHC = CH // 2
    RP = NPAD // NS

    @functools.partial(
        pl.kernel,
        out_type=jax.ShapeDtypeStruct((NC, NPAD), jnp.float32),
        mesh=_mesh(),
        compiler_params=pltpu.CompilerParams(needs_layout_passes=False),
        scratch_types=[
            pltpu.VMEM_SHARED((NS, NPAD), jnp.float32),
            pltpu.VMEM((NPAD,), jnp.float32),
            pltpu.VMEM((CHC, LB), jnp.int32),
            pltpu.VMEM((NS, RP), jnp.float32),
        ],
    )
    def deg_kernel(dst3, z1_h, out, slab, cnt, dstv, redv):
        c = lax.axis_index("c")
        s = lax.axis_index("s")
        pltpu.sync_copy(z1_h, cnt)
        pltpu.sync_copy(dst3.at[s, pl.ds(c * CHC, CHC)], dstv)
        one16 = jnp.ones((16,), jnp.float32)

        def body(j, carry):
            for k in range(LB // 16):
                idx = dstv[j, pl.ds(16 * k, 16)]
                plsc.addupdate_scatter(cnt, [idx], one16)
            return carry

        lax.fori_loop(0, CHC, body, 0)
        pltpu.sync_copy(cnt, slab.at[s])
        plsc.subcore_barrier()
        pltpu.sync_copy(slab.at[:, pl.ds(s * RP, RP)], redv)

        def red(v, carry):
            acc = redv[0, pl.ds(16 * v, 16)]
            for t in range(1, NS):
                acc = acc + redv[t, pl.ds(16 * v, 16)]
            redv[0, pl.ds(16 * v, 16)] = acc
            return carry

        lax.fori_loop(0, RP // 16, red, 0)
        pltpu.sync_copy(redv.at[0], out.at[c, pl.ds(s * RP, RP)])

    return deg_kernel


def _make_scatter(NPAD, CH, W, split_edges):
    RP = NPAD // NS
    GI = 8
    CHW = CH // 2 if split_edges else CH
    NG = CHW // GI

    @functools.partial(
        pl.kernel,
        out_type=jax.ShapeDtypeStruct((NC, NPAD, W), jnp.float32),
        mesh=_mesh(),
        scratch_types=[
            pltpu.VMEM_SHARED((NPAD, W), jnp.float32),
            pltpu.VMEM((GI, LB), jnp.int32),
            pltpu.VMEM((GI, LB), jnp.int32),
            pltpu.VMEM((GI, LB), jnp.int32),
            pltpu.VMEM((GI, LB), jnp.int32),
            pltpu.VMEM((LB, W), jnp.float32),
            pltpu.VMEM((LB, W), jnp.float32),
            pltpu.SemaphoreType.DMA,
            pltpu.SemaphoreType.DMA,
            pltpu.SemaphoreType.DMA,
            pltpu.SemaphoreType.DMA,
        ],
    )
    def scat_kernel(table, src_h, dst_h, zeros_h, out, agg, sv0, sv1, dv0,
                    dv1, rb0, rb1, gs0, gs1, is0, is1):
        c = lax.axis_index("c")
        s = lax.axis_index("s")
        rbs = (rb0, rb1)
        gss = (gs0, gs1)
        slots = ((sv0, dv0, is0), (sv1, dv1, is1))

        def src_slice(g):
            if split_edges:
                return src_h.at[s, pl.ds(c * CHW + g * GI, GI)]
            return src_h.at[c, s, pl.ds(g * GI, GI)]

        def dst_slice(g):
            if split_edges:
                return dst_h.at[s, pl.ds(c * CHW + g * GI, GI)]
            return dst_h.at[s, pl.ds(g * GI, GI)]

        pltpu.sync_copy(zeros_h.at[pl.ds(s * RP, RP)],
                        agg.at[pl.ds(s * RP, RP)])
        plsc.subcore_barrier()

        def iload(g, slot):
            sv, dv, sem = slot
            pltpu.async_copy(src_slice(g), sv, sem)
            pltpu.async_copy(dst_slice(g), dv, sem)

        def iwait(slot):
            sv, dv, sem = slot
            pltpu.make_async_copy(src_slice(0), sv, sem).wait()
            pltpu.make_async_copy(dst_slice(0), dv, sem).wait()

        def gstart(sv, k, rb, sem):
            pltpu.async_copy(table.at[sv.at[k]], rb, sem)

        def gwait(rb, sem):
            pltpu.make_async_copy(table.at[sv0.at[0]], rb, sem).wait()

        def scat(dv, k, rb):
            pltpu.sync_copy(rb, agg.at[dv.at[k]], add=True)

        iload(0, slots[0])
        iwait(slots[0])
        gstart(sv0, 0, rb0, gs0)

        def group(g, p, has_next):
            sv, dv, _ = slots[p]
            nxt = slots[1 - p]
            for k in range(GI):
                par = k % 2
                if k < GI - 1:
                    gstart(sv, k + 1, rbs[1 - par], gss[1 - par])
                else:
                    @pl.when(has_next)
                    def _():
                        iwait(nxt)
                        gstart(nxt[0], 0, rbs[1 - par], gss[1 - par])
                gwait(rbs[par], gss[par])
                scat(dv, k, rbs[par])

        def body(u, carry):
            g0 = 2 * u
            iload(g0 + 1, slots[1])
            group(g0, 0, True)

            @pl.when(u + 1 < NG // 2)
            def _():
                iload(g0 + 2, slots[0])

            group(g0 + 1, 1, u + 1 < NG // 2)
            return carry

        lax.fori_loop(0, NG // 2, body, 0)
        plsc.subcore_barrier()
        pltpu.sync_copy(agg.at[pl.ds(s * RP, RP)],
                        out.at[c, pl.ds(s * RP, RP)])

    return scat_kernel


def _dinv_of(degp_ref):
    deg = degp_ref[0, :, 0:1] + degp_ref[1, :, 0:1] + 1.0
    return lax.rsqrt(jnp.maximum(deg, 1.0))


def _make_tc1(NPAD, F, H, BR):
    def body(x_ref, w1_ref, b1_ref, a1_ref, w2_ref, b2_ref, wg1_ref,
             degp_ref, hs_ref):
        dinv = _dinv_of(degp_ref)
        t = jnp.dot(x_ref[...], w1_ref[...],
                    preferred_element_type=jnp.float32) + b1_ref[...]
        a1 = a1_ref[0, 0]
        t = jnp.where(t >= 0, t, a1 * t)
        x1 = jnp.dot(t, w2_ref[...],
                     preferred_element_type=jnp.float32) + b2_ref[...]
        h = jnp.dot(x1, wg1_ref[...], preferred_element_type=jnp.float32)
        hs = h * dinv
        hs_ref[0] = hs[:, : H // 2]
        hs_ref[1] = hs[:, H // 2:]

    return pl.pallas_call(
        body,
        grid=(NPAD // BR,),
        in_specs=[
            pl.BlockSpec((BR, F), lambda i: (i, 0)),
            pl.BlockSpec((F, H), lambda i: (0, 0)),
            pl.BlockSpec((1, H), lambda i: (0, 0)),
            pl.BlockSpec(memory_space=pltpu.SMEM),
            pl.BlockSpec((H, F), lambda i: (0, 0)),
            pl.BlockSpec((1, F), lambda i: (0, 0)),
            pl.BlockSpec((F, H), lambda i: (0, 0)),
            pl.BlockSpec((NC, BR, 1), lambda i: (0, i, 0)),
        ],
        out_specs=pl.BlockSpec((NC, BR, H // 2), lambda i: (0, i, 0)),
        out_shape=jax.ShapeDtypeStruct((NC, NPAD, H // 2), jnp.float32),
    )


def _make_tc2(NPAD, F, H, BR):
    def body(pre_ref, hs_ref, degp_ref, bg1_ref, ag_ref, wg2_ref, out_ref):
        dinv = _dinv_of(degp_ref)
        t = jnp.concatenate(
            [pre_ref[0] + hs_ref[0], pre_ref[1] + hs_ref[1]], axis=1)
        t = dinv * t + bg1_ref[...]
        ag = ag_ref[0, 0]
        x2 = jnp.where(t >= 0, t, ag * t)
        h2 = jnp.dot(x2, wg2_ref[...], preferred_element_type=jnp.float32)
        out_ref[...] = h2 * dinv

    return pl.pallas_call(
        body,
        grid=(NPAD // BR,),
        in_specs=[
            pl.BlockSpec((NC, BR, H // 2), lambda i: (0, i, 0)),
            pl.BlockSpec((NC, BR, H // 2), lambda i: (0, i, 0)),
            pl.BlockSpec((NC, BR, 1), lambda i: (0, i, 0)),
            pl.BlockSpec((1, H), lambda i: (0, 0)),
            pl.BlockSpec(memory_space=pltpu.SMEM),
            pl.BlockSpec((H, F), lambda i: (0, 0)),
        ],
        out_specs=pl.BlockSpec((BR, F), lambda i: (i, 0)),
        out_shape=jax.ShapeDtypeStruct((NPAD, F), jnp.float32),
    )


def _make_tc3(NPAD, F, H, BR):
    def body(pre_ref, hs_ref, degp_ref, bg2_ref, out_ref):
        dinv = _dinv_of(degp_ref)
        t = pre_ref[0] + pre_ref[1] + hs_ref[...]
        out_ref[...] = dinv * t + bg2_ref[...]

    return pl.pallas_call(
        body,
        grid=(NPAD // BR,),
        in_specs=[
            pl.BlockSpec((NC, BR, F), lambda i: (0, i, 0)),
            pl.BlockSpec((BR, F), lambda i: (i, 0)),
            pl.BlockSpec((NC, BR, 1), lambda i: (0, i, 0)),
            pl.BlockSpec((1, F), lambda i: (0, 0)),
        ],
        out_specs=pl.BlockSpec((BR, F), lambda i: (i, 0)),
        out_shape=jax.ShapeDtypeStruct((NPAD, F), jnp.float32),
    )


def kernel(X, edge_index, W1, b1, a1, W2, b2, Wg1, bg1, Wg2, bg2, ag):
    B, n, F = X.shape
    H = W1.shape[1]
    e = edge_index.shape[1]
    BR = 512
    NPAD = -(-n // (NS * BR // 8)) * (NS * BR // 8)
    CH = -(-e // (NS * LB * 32)) * 32
    EP = NS * CH * LB

    src = edge_index[0].astype(jnp.int32)
    dst = edge_index[1].astype(jnp.int32)
    src_p = jnp.concatenate([src, jnp.zeros((EP - e,), jnp.int32)])
    dst_p = jnp.concatenate([dst, jnp.full((EP - e,), NPAD - 1, jnp.int32)])
    src2 = src_p.reshape(NS, CH, LB)
    src3 = jnp.stack([src2, src2 + NPAD])
    dst3 = dst_p.reshape(NS, CH, LB)

    x0 = jnp.pad(X.reshape(n, F), ((0, NPAD - n), (0, 0)))
    z1 = jnp.zeros((NPAD,), jnp.float32)
    zH = jnp.zeros((NPAD, H // 2), jnp.float32)
    zF = jnp.zeros((NPAD, F), jnp.float32)

    degp = _make_deg(NPAD, CH)(dst3, z1).reshape(NC, NPAD, 1)
    hs1 = _make_tc1(NPAD, F, H, BR)(
        x0, W1, b1.reshape(1, H), a1.reshape(1, 1), W2, b2.reshape(1, F),
        Wg1, degp)
    pre1 = _make_scatter(NPAD, CH, H // 2, False)(
        hs1.reshape(NC * NPAD, H // 2), src3, dst3, zH)
    hs2 = _make_tc2(NPAD, F, H, BR)(
        pre1, hs1, degp, bg1.reshape(1, H), ag.reshape(1, 1), Wg2)
    pre2 = _make_scatter(NPAD, CH, F, True)(hs2, src2, dst3, zF)
    y = _make_tc3(NPAD, F, H, BR)(pre2, hs2, degp, bg2.reshape(1, F))
    return y[:n].reshape(B, n, F)
